# baseline (speedup 1.0000x reference)
import dataclasses
import functools

import jax
import jax.numpy as jnp
from jax import lax
from jax.experimental import pallas as pl
from jax.experimental.pallas import tpu as pltpu
from jax.experimental.pallas import tpu_sc as plsc

_CP = pltpu.CompilerParams()
if "needs_layout_passes" in pltpu.CompilerParams.__dataclass_fields__:
    _CP = dataclasses.replace(_CP, needs_layout_passes=False)

_B, _F, _V, _K = 4096, 26, 100000, 32
_NC, _NS = 2, 16
_VSEG = 1536
_NSEG = 66
_FLMAX = 14
_TMAX = (_NSEG + _NS - 1) // _NS
_TS0 = 98560


def _body(tt_hbm, tailp_hbm, inpT_hbm, out_hbm,
          ridx_t, fidx_t, vlist, slabs, h1_pk, hb_pk,
          hbufs, sibufs, ring, sems, ssem, vsem):
    c = lax.axis_index("c")
    s = lax.axis_index("s")
    iota16 = lax.broadcasted_iota(jnp.int32, (16,), 0)

    fl_n = jnp.where(c == 0, 12, 14)
    f_base = jnp.where(c == 0, 0, 12)
    dump = _F * _B + c * _NS + s

    ring[0] = 0
    for j in range(4):
        ring[1 + j] = 0

    @pl.loop(0, _F)
    def _gen(f):
        ridx_t[f, pl.ds(0, 16)] = f * _K + iota16
        ridx_t[f, pl.ds(16, 16)] = f * _K + 16 + iota16

    def fetch(gf, t, buf):
        seg = s + t * _NS
        s0 = pl.multiple_of(seg * _VSEG, 128)

        @pl.when(seg < _NSEG - 1)
        def _():
            pltpu.async_copy(
                tt_hbm.at[ridx_t.at[gf], pl.ds(s0, _VSEG)],
                slabs.at[buf],
                sems.at[buf],
            )

        @pl.when(seg == _NSEG - 1)
        def _():
            pltpu.async_copy(
                tt_hbm.at[ridx_t.at[gf], pl.ds(pl.multiple_of(_TS0, 128), 1408)],
                slabs.at[buf].at[:, pl.ds(0, 1408)],
                sems.at[buf],
            )
            pltpu.async_copy(
                tailp_hbm.at[ridx_t.at[gf], pl.ds(0, 128)],
                slabs.at[buf].at[:, pl.ds(1408, 128)],
                sems.at[buf],
            )

    def wait_slab(buf):
        pltpu.make_async_copy(
            tt_hbm.at[ridx_t.at[0], pl.ds(0, _VSEG)],
            slabs.at[buf],
            sems.at[buf],
        ).wait()

    def issue_vlist(gf):
        fidx_t[pl.ds(0, 16)] = jnp.broadcast_to(gf, (16,))
        pltpu.async_copy(inpT_hbm.at[fidx_t.at[pl.ds(0, 1)]], vlist, vsem)

    def wait_vlist():
        pltpu.make_async_copy(
            inpT_hbm.at[fidx_t.at[pl.ds(0, 1)]], vlist, vsem
        ).wait()

    def ring_wait(bi):
        @pl.when(ring[1 + bi] == 1)
        def _():
            pltpu.make_async_copy(
                hbufs.at[bi],
                out_hbm.at[sibufs.at[bi]],
                ssem.at[bi],
            ).wait()

    def process_seg(fl, seg, n1, buf):
        s0 = seg * _VSEG
        s0c = jnp.where(seg == _NSEG - 1, _TS0, s0)

        def scan2(j, off):
            pk = h1_pk[pl.ds(j * 16, 16)]
            vv = pk & 131071
            m = (vv >= s0) & (vv < s0 + _VSEG)
            cnt = plsc.all_reduce_population_count(m)[0]

            @pl.when(cnt > 0)
            def _():
                pk2 = ((pk >> 17) << 11) | (vv - s0c)
                plsc.store_compressed(hb_pk.at[pl.ds(off, 16)], pk2, mask=m)

            return off + cnt

        n2 = lax.fori_loop(0, 17, scan2, 0, unroll=4)
        n2 = lax.fori_loop(17, (n1 + 15) >> 4, scan2, n2)
        hb_pk[pl.ds(n2, 16)] = jnp.broadcast_to(_B << 11, (16,))

        wait_slab(buf)

        def batch_step(g, _):
            pk2 = hb_pk[pl.ds(g * 16, 16)]
            bb = pk2 >> 11
            vv = pk2 & 2047
            sidx = jnp.where(bb >= _B, dump, bb * _F + f_base + fl)
            bi = ring[0] & 3
            ring_wait(bi)
            sibufs[bi, pl.ds(0, 16)] = sidx
            for lane in range(16):
                col = jnp.broadcast_to(vv[lane], (16,))
                lo = plsc.load_gather(slabs.at[buf], [iota16, col])
                hi = plsc.load_gather(slabs.at[buf], [iota16 + 16, col])
                hbufs[bi, lane, pl.ds(0, 16)] = lo
                hbufs[bi, lane, pl.ds(16, 16)] = hi

            pltpu.async_copy(hbufs.at[bi], out_hbm.at[sibufs.at[bi]],
                             ssem.at[bi])

            ring[1 + bi] = 1
            ring[0] = ring[0] + 1
            return 0

        lax.fori_loop(0, (n2 + 15) >> 4, batch_step, 0)

    issue_vlist(f_base)

    @pl.loop(0, _FLMAX)
    def _field(fl):
        @pl.when(fl < fl_n)
        def _():
            gf = f_base + fl
            fetch(gf, 0, 0)
            wait_vlist()

            def scan1(j, off):
                x = vlist[0, pl.ds(j * 16, 16)]
                seg_of = ((x >> 9) * 21846) >> 16
                m = (seg_of & (_NS - 1)) == s
                cnt = plsc.all_reduce_population_count(m)[0]

                @pl.when(cnt > 0)
                def _():
                    plsc.store_compressed(h1_pk.at[pl.ds(off, 16)],
                                          ((j * 16 + iota16) << 17) | x,
                                          mask=m)

                return off + cnt

            @pl.loop(0, 18)
            def _prefill(j):
                h1_pk[pl.ds(j * 16, 16)] = jnp.broadcast_to(
                    jnp.int32((_B << 17) | 131071), (16,))

            n1 = lax.fori_loop(0, _B // 16, scan1, 0, unroll=4)
            h1_pk[pl.ds(n1, 16)] = jnp.broadcast_to(
                jnp.int32((_B << 17) | 131071), (16,))

            @pl.when(fl + 1 < fl_n)
            def _():
                issue_vlist(gf + 1)

            @pl.loop(0, _TMAX)
            def _t(t):
                seg = s + t * _NS

                @pl.when(seg < _NSEG)
                def _():
                    @pl.when(seg + _NS < _NSEG)
                    def _():
                        fetch(gf, t + 1, (t + 1) & 1)

                    process_seg(fl, seg, n1, t & 1)

    for j in range(4):

        @pl.when(ring[1 + j] == 1)
        def _(j=j):
            pltpu.make_async_copy(
                hbufs.at[j],
                out_hbm.at[sibufs.at[j]],
                ssem.at[j],
            ).wait()


@jax.jit
def kernel(inputs, tables):
    f, voc, k = tables.shape
    b = inputs.shape[0]
    tt = jnp.transpose(tables, (0, 2, 1)).reshape(f * k, voc)
    tailp = jnp.pad(tt[:, _TS0 + 1408 :], ((0, 0), (0, 96)))
    inpT = jnp.transpose(inputs)
    mesh = plsc.VectorSubcoreMesh(core_axis_name="c", subcore_axis_name="s")
    run = functools.partial(
        pl.kernel,
        mesh=mesh,
        out_type=jax.ShapeDtypeStruct((b * f + 32, 128), jnp.float32),
        compiler_params=_CP,
        scratch_types=[
            pltpu.VMEM((_F, _K), jnp.int32),
            pltpu.VMEM((16,), jnp.int32),
            pltpu.VMEM((1, _B), jnp.int32),
            pltpu.VMEM((2, _K, _VSEG), jnp.float32),
            pltpu.VMEM((_B + 16,), jnp.int32),
            pltpu.VMEM((_B + 16,), jnp.int32),
            pltpu.VMEM((4, 16, 128), jnp.float32),
            pltpu.VMEM((4, 16), jnp.int32),
            pltpu.SMEM((16,), jnp.int32),
            pltpu.SemaphoreType.DMA((2,)),
            pltpu.SemaphoreType.DMA((4,)),
            pltpu.SemaphoreType.DMA,
        ],
    )(_body)
    out2 = run(tt, tailp, inpT)
    return out2[: b * f, :k].reshape(b, f, k).reshape(b, f * k)
